# Initial kernel scaffold; baseline (speedup 1.0000x reference)
import jax
import jax.numpy as jnp
from jax.experimental import pallas as pl


def kernel(X, node_idx, hyedge_idx, theta, bias):
    raise NotImplementedError("write your pallas kernel here")


# speedup vs baseline: 3.8945x; 3.8945x over previous
import functools

import jax
import jax.numpy as jnp
from jax import lax
from jax.experimental import pallas as pl
from jax.experimental.pallas import tpu as pltpu
from jax.experimental.pallas import tpu_sc as plsc

N_NODES = 10000
N_HYEDGES = 5000
N_INC = 320000
D = 128

NC = 2
NS = 16
NW = NC * NS
PER_W = N_INC // NW
CH = 128
NFULL = PER_W // CH
TAIL = PER_W - NFULL * CH

Y_PAD = 5120
X_PAD = 10240


def _make_sc_pass(n_dst_pad):
    rows_per_tile = n_dst_pad // NS
    mesh = plsc.VectorSubcoreMesh(core_axis_name="c", subcore_axis_name="s")

    @functools.partial(
        pl.kernel,
        out_type=jax.ShapeDtypeStruct((NC, n_dst_pad, D), jnp.float32),
        mesh=mesh,
        scratch_types=[
            pltpu.VMEM((16, D), jnp.float32),
            pltpu.VMEM((CH,), jnp.int32),
            pltpu.VMEM((CH,), jnp.int32),
            pltpu.VMEM((CH, D), jnp.float32),
            pltpu.VMEM((TAIL,), jnp.int32),
            pltpu.VMEM((TAIL,), jnp.int32),
            pltpu.VMEM((TAIL, D), jnp.float32),
            pltpu.VMEM_SHARED((n_dst_pad, D), jnp.float32),
            pltpu.SemaphoreType.DMA,
        ],
    )
    def sc_pass(src_hbm, gidx_hbm, sidx_hbm, out_hbm,
                zb, gi, si, rows, git, sit, rowst, acc, sem):
        c = lax.axis_index("c")
        s = lax.axis_index("s")
        wid = s * NC + c
        base = wid * PER_W

        def zrow(i, _):
            def zcol(j, _):
                zb[i, pl.ds(j * 16, 16)] = jnp.zeros((16,), jnp.float32)
                return 0
            return lax.fori_loop(0, D // 16, zcol, 0)
        lax.fori_loop(0, 16, zrow, 0)

        slab = s * rows_per_tile

        def zslab(k, _):
            pltpu.sync_copy(zb, acc.at[pl.ds(slab + k * 16, 16)])
            return 0
        lax.fori_loop(0, rows_per_tile // 16, zslab, 0)
        plsc.subcore_barrier()

        def chunk(j, _):
            off = base + j * CH
            pltpu.sync_copy(gidx_hbm.at[pl.ds(off, CH)], gi)
            pltpu.async_copy(src_hbm.at[gi], rows, sem).wait()
            pltpu.sync_copy(sidx_hbm.at[pl.ds(off, CH)], si)
            pltpu.sync_copy(rows, acc.at[si], add=True)
            return 0
        lax.fori_loop(0, NFULL, chunk, 0)

        toff = base + NFULL * CH
        pltpu.sync_copy(gidx_hbm.at[pl.ds(toff, TAIL)], git)
        pltpu.async_copy(src_hbm.at[git], rowst, sem).wait()
        pltpu.sync_copy(sidx_hbm.at[pl.ds(toff, TAIL)], sit)
        pltpu.sync_copy(rowst, acc.at[sit], add=True)

        plsc.subcore_barrier()
        pltpu.sync_copy(acc.at[pl.ds(slab, rows_per_tile)],
                        out_hbm.at[c, pl.ds(slab, rows_per_tile)])

    return sc_pass


_sc_pass_Y = _make_sc_pass(Y_PAD)
_sc_pass_X = _make_sc_pass(X_PAD)


def _mm_body(x_ref, t_ref, o_ref):
    o_ref[...] = jnp.dot(x_ref[...], t_ref[...],
                         preferred_element_type=jnp.float32)


def _matmul(X, theta):
    return pl.pallas_call(
        _mm_body,
        grid=(10,),
        in_specs=[pl.BlockSpec((1000, D), lambda i: (i, 0)),
                  pl.BlockSpec((D, D), lambda i: (0, 0))],
        out_specs=pl.BlockSpec((1000, D), lambda i: (i, 0)),
        out_shape=jax.ShapeDtypeStruct((N_NODES, D), jnp.float32),
    )(X, theta)


def _sum2_body(p_ref, o_ref):
    o_ref[...] = p_ref[0] + p_ref[1]


def _sum_partials(P):
    n = P.shape[1]
    blk = n // 10
    return pl.pallas_call(
        _sum2_body,
        grid=(10,),
        in_specs=[pl.BlockSpec((NC, blk, D), lambda i: (0, i, 0))],
        out_specs=pl.BlockSpec((blk, D), lambda i: (i, 0)),
        out_shape=jax.ShapeDtypeStruct((n, D), jnp.float32),
    )(P)


def _sum2b_body(p_ref, b_ref, o_ref):
    o_ref[...] = p_ref[0] + p_ref[1] + b_ref[...]


def _sum_partials_bias(P, bias2d):
    n = P.shape[1]
    blk = n // 10
    return pl.pallas_call(
        _sum2b_body,
        grid=(10,),
        in_specs=[pl.BlockSpec((NC, blk, D), lambda i: (0, i, 0)),
                  pl.BlockSpec((1, D), lambda i: (0, 0))],
        out_specs=pl.BlockSpec((blk, D), lambda i: (i, 0)),
        out_shape=jax.ShapeDtypeStruct((n, D), jnp.float32),
    )(P, bias2d)


def kernel(X, node_idx, hyedge_idx, theta, bias):
    ni = node_idx.astype(jnp.int32)
    he = hyedge_idx.astype(jnp.int32)
    Xp = _matmul(X, theta)
    Yp = _sc_pass_Y(Xp, ni, he)
    Y = _sum_partials(Yp)
    Xnp = _sc_pass_X(Y, he, ni)
    Xn = _sum_partials_bias(Xnp, bias.reshape(1, D))
    return Xn[:N_NODES]


# speedup vs baseline: 6.2752x; 1.6113x over previous
import functools

import jax
import jax.numpy as jnp
from jax import lax
from jax.experimental import pallas as pl
from jax.experimental.pallas import tpu as pltpu
from jax.experimental.pallas import tpu_sc as plsc

N_NODES = 10000
N_HYEDGES = 5000
N_INC = 320000
D = 128

NC = 2
NS = 16
NW = NC * NS
CH = 128
NCHUNK = N_INC // CH
NFULL = NCHUNK // NW
NREST = NCHUNK - NFULL * NW

Y_PAD = 5120
X_PAD = 10240


def _make_sc_pass(n_dst_pad):
    rows_per_tile = n_dst_pad // NS
    mesh = plsc.VectorSubcoreMesh(core_axis_name="c", subcore_axis_name="s")

    @functools.partial(
        pl.kernel,
        out_type=jax.ShapeDtypeStruct((NC, n_dst_pad, D), jnp.float32),
        mesh=mesh,
        scratch_types=[
            pltpu.VMEM((16, D), jnp.float32),
            pltpu.VMEM((2, CH), jnp.int32),
            pltpu.VMEM((2, CH), jnp.int32),
            pltpu.VMEM((CH, D), jnp.float32),
            pltpu.VMEM((CH, D), jnp.float32),
            pltpu.VMEM_SHARED((n_dst_pad, D), jnp.float32),
            pltpu.SemaphoreType.DMA,
            pltpu.SemaphoreType.DMA,
            pltpu.SemaphoreType.DMA,
            pltpu.SemaphoreType.DMA,
        ],
    )
    def sc_pass(src_hbm, gidx_hbm, sidx_hbm, out_hbm,
                zb, gi, si, rows0, rows1, acc, isem0, isem1, gsem0, gsem1):
        c = lax.axis_index("c")
        s = lax.axis_index("s")
        wid = s * NC + c

        def zrow(i, _):
            def zcol(j, _):
                zb[i, pl.ds(j * 16, 16)] = jnp.zeros((16,), jnp.float32)
                return 0
            return lax.fori_loop(0, D // 16, zcol, 0)
        lax.fori_loop(0, 16, zrow, 0)

        slab = s * rows_per_tile

        def zslab(k, _):
            pltpu.sync_copy(zb, acc.at[pl.ds(slab + k * 16, 16)])
            return 0
        lax.fori_loop(0, rows_per_tile // 16, zslab, 0)
        plsc.subcore_barrier()

        def chunk_of(j):
            return jnp.minimum(wid + NW * j, NCHUNK - 1)

        g0 = chunk_of(0)
        pltpu.async_copy(gidx_hbm.at[g0], gi.at[0], isem0)
        pltpu.async_copy(sidx_hbm.at[g0], si.at[0], isem0)
        pltpu.make_async_copy(gidx_hbm.at[g0], gi.at[0], isem0).wait()
        pltpu.make_async_copy(sidx_hbm.at[g0], si.at[0], isem0).wait()
        pltpu.async_copy(src_hbm.at[gi.at[0]], rows0, gsem0)
        g1 = chunk_of(1)
        pltpu.async_copy(gidx_hbm.at[g1], gi.at[1], isem1)
        pltpu.async_copy(sidx_hbm.at[g1], si.at[1], isem1)

        def body(j2, _):
            j = j2 * 2
            ga = chunk_of(j + 1)
            gb = chunk_of(j + 2)
            gc = chunk_of(j + 3)
            pltpu.make_async_copy(gidx_hbm.at[ga], gi.at[1], isem1).wait()
            pltpu.make_async_copy(sidx_hbm.at[ga], si.at[1], isem1).wait()
            pltpu.async_copy(src_hbm.at[gi.at[1]], rows1, gsem1)
            pltpu.make_async_copy(src_hbm.at[gi.at[0]], rows0, gsem0).wait()
            pltpu.sync_copy(rows0, acc.at[si.at[0]], add=True)
            pltpu.async_copy(gidx_hbm.at[gb], gi.at[0], isem0)
            pltpu.async_copy(sidx_hbm.at[gb], si.at[0], isem0)
            pltpu.make_async_copy(gidx_hbm.at[gb], gi.at[0], isem0).wait()
            pltpu.make_async_copy(sidx_hbm.at[gb], si.at[0], isem0).wait()
            pltpu.async_copy(src_hbm.at[gi.at[0]], rows0, gsem0)
            pltpu.make_async_copy(src_hbm.at[gi.at[1]], rows1, gsem1).wait()
            pltpu.sync_copy(rows1, acc.at[si.at[1]], add=True)
            pltpu.async_copy(gidx_hbm.at[gc], gi.at[1], isem1)
            pltpu.async_copy(sidx_hbm.at[gc], si.at[1], isem1)
            return 0

        lax.fori_loop(0, NFULL // 2, body, 0)

        gd = chunk_of(NFULL)
        pltpu.make_async_copy(src_hbm.at[gi.at[0]], rows0, gsem0).wait()
        pltpu.make_async_copy(gidx_hbm.at[gd], gi.at[1], isem1).wait()
        pltpu.make_async_copy(sidx_hbm.at[gd], si.at[1], isem1).wait()

        @pl.when(wid < NREST)
        def _():
            gl = NCHUNK - NREST + wid
            pltpu.sync_copy(gidx_hbm.at[gl], gi.at[0])
            pltpu.sync_copy(sidx_hbm.at[gl], si.at[0])
            pltpu.async_copy(src_hbm.at[gi.at[0]], rows0, gsem0).wait()
            pltpu.sync_copy(rows0, acc.at[si.at[0]], add=True)

        plsc.subcore_barrier()
        pltpu.sync_copy(acc.at[pl.ds(slab, rows_per_tile)],
                        out_hbm.at[c, pl.ds(slab, rows_per_tile)])

    return sc_pass


_sc_pass_Y = _make_sc_pass(Y_PAD)
_sc_pass_X = _make_sc_pass(X_PAD)


def _mm_body(x_ref, t_ref, o_ref):
    o_ref[...] = jnp.dot(x_ref[...], t_ref[...],
                         preferred_element_type=jnp.float32)


def _matmul(X, theta):
    return pl.pallas_call(
        _mm_body,
        grid=(10,),
        in_specs=[pl.BlockSpec((1000, D), lambda i: (i, 0)),
                  pl.BlockSpec((D, D), lambda i: (0, 0))],
        out_specs=pl.BlockSpec((1000, D), lambda i: (i, 0)),
        out_shape=jax.ShapeDtypeStruct((N_NODES, D), jnp.float32),
    )(X, theta)


def _sum2_body(p_ref, o_ref):
    o_ref[...] = p_ref[0] + p_ref[1]


def _sum_partials(P):
    n = P.shape[1]
    blk = n // 10
    return pl.pallas_call(
        _sum2_body,
        grid=(10,),
        in_specs=[pl.BlockSpec((NC, blk, D), lambda i: (0, i, 0))],
        out_specs=pl.BlockSpec((blk, D), lambda i: (i, 0)),
        out_shape=jax.ShapeDtypeStruct((n, D), jnp.float32),
    )(P)


def _sum2b_body(p_ref, b_ref, o_ref):
    o_ref[...] = p_ref[0] + p_ref[1] + b_ref[...]


def _sum_partials_bias(P, bias2d):
    n = P.shape[1]
    blk = n // 10
    return pl.pallas_call(
        _sum2b_body,
        grid=(10,),
        in_specs=[pl.BlockSpec((NC, blk, D), lambda i: (0, i, 0)),
                  pl.BlockSpec((1, D), lambda i: (0, 0))],
        out_specs=pl.BlockSpec((blk, D), lambda i: (i, 0)),
        out_shape=jax.ShapeDtypeStruct((n, D), jnp.float32),
    )(P, bias2d)


def kernel(X, node_idx, hyedge_idx, theta, bias):
    ni = node_idx.astype(jnp.int32).reshape(NCHUNK, CH)
    he = hyedge_idx.astype(jnp.int32).reshape(NCHUNK, CH)
    Xp = _matmul(X, theta)
    Yp = _sc_pass_Y(Xp, ni, he)
    Y = _sum_partials(Yp)
    Xnp = _sc_pass_X(Y, he, ni)
    Xn = _sum_partials_bias(Xnp, bias.reshape(1, D))
    return Xn[:N_NODES]


# speedup vs baseline: 7.6474x; 1.2187x over previous
import functools

import jax
import jax.numpy as jnp
from jax import lax
from jax.experimental import pallas as pl
from jax.experimental.pallas import tpu as pltpu
from jax.experimental.pallas import tpu_sc as plsc

N_NODES = 10000
N_HYEDGES = 5000
N_INC = 320000
D = 128

NC = 2
NS = 16
NW = NC * NS
CH = 64
NCHUNK = N_INC // CH
NFULL = NCHUNK // NW
NREST = NCHUNK - NFULL * NW
NR = 4
NI = 8
PEEL = 4

Y_PAD = 5120
X_PAD = 10240


def _make_sc_pass(n_dst_pad):
    rows_per_tile = n_dst_pad // NS
    mesh = plsc.VectorSubcoreMesh(core_axis_name="c", subcore_axis_name="s")

    @functools.partial(
        pl.kernel,
        out_type=jax.ShapeDtypeStruct((NC, n_dst_pad, D), jnp.float32),
        mesh=mesh,
        scratch_types=(
            [pltpu.VMEM((16, D), jnp.float32),
             pltpu.VMEM((NI, CH), jnp.int32),
             pltpu.VMEM((NI, CH), jnp.int32)]
            + [pltpu.VMEM((CH, D), jnp.float32) for _ in range(NR)]
            + [pltpu.VMEM_SHARED((n_dst_pad, D), jnp.float32)]
            + [pltpu.SemaphoreType.DMA for _ in range(NI + 2 * NR)]
        ),
    )
    def sc_pass(src_hbm, gidx_hbm, sidx_hbm, out_hbm, zb, gi, si, *rest):
        rows = rest[:NR]
        acc = rest[NR]
        isem = rest[NR + 1:NR + 1 + NI]
        gsem = rest[NR + 1 + NI:NR + 1 + NI + NR]
        ssem = rest[NR + 1 + NI + NR:NR + 1 + NI + 2 * NR]
        c = lax.axis_index("c")
        s = lax.axis_index("s")
        wid = s * NC + c

        def zrow(i, _):
            def zcol(j, _):
                zb[i, pl.ds(j * 16, 16)] = jnp.zeros((16,), jnp.float32)
                return 0
            return lax.fori_loop(0, D // 16, zcol, 0)
        lax.fori_loop(0, 16, zrow, 0)

        slab = s * rows_per_tile

        def zslab(k, _):
            pltpu.sync_copy(zb, acc.at[pl.ds(slab + k * 16, 16)])
            return 0
        lax.fori_loop(0, rows_per_tile // 16, zslab, 0)
        plsc.subcore_barrier()

        def chunk_of(j):
            return jnp.minimum(wid + NW * j, NCHUNK - 1)

        def idx_issue(j, sl):
            g = chunk_of(j)
            pltpu.async_copy(gidx_hbm.at[g], gi.at[sl], isem[sl])
            pltpu.async_copy(sidx_hbm.at[g], si.at[sl], isem[sl])

        def idx_wait(j, sl):
            g = chunk_of(j)
            pltpu.make_async_copy(gidx_hbm.at[g], gi.at[sl], isem[sl]).wait()
            pltpu.make_async_copy(sidx_hbm.at[g], si.at[sl], isem[sl]).wait()

        def gather_issue(rs, isl):
            pltpu.async_copy(src_hbm.at[gi.at[isl]], rows[rs], gsem[rs])

        def gather_wait(rs, isl):
            pltpu.make_async_copy(src_hbm.at[gi.at[isl]], rows[rs],
                                  gsem[rs]).wait()

        def scatter_issue(rs, isl):
            pltpu.async_copy(rows[rs], acc.at[si.at[isl]], ssem[rs], add=True)

        def scatter_wait(rs, isl):
            pltpu.make_async_copy(rows[rs], acc.at[si.at[isl]],
                                  ssem[rs]).wait()

        def steps(j, u):
            if u >= NR:
                scatter_wait(u % NR, (u + NI - NR) % NI)
            idx_issue(j + PEEL, (u + PEEL) % NI)
            idx_wait(j, u % NI)
            gather_issue(u % NR, u % NI)
            if u >= 2:
                gather_wait((u - 2) % NR, (u - 2) % NI)
                scatter_issue((u - 2) % NR, (u - 2) % NI)

        for jj in range(PEEL):
            idx_issue(jj, jj)

        for jj in range(PEEL):
            steps(jj, jj)

        def body(k, _):
            j0 = PEEL + k * NI
            for v in range(NI):
                steps(j0 + v, PEEL + v)
            return 0

        lax.fori_loop(0, (NFULL - PEEL) // NI, body, 0)

        gather_wait((NFULL - 2) % NR, (NFULL - 2) % NI)
        scatter_issue((NFULL - 2) % NR, (NFULL - 2) % NI)
        gather_wait((NFULL - 1) % NR, (NFULL - 1) % NI)
        scatter_issue((NFULL - 1) % NR, (NFULL - 1) % NI)
        for dd in range(NFULL - NR, NFULL):
            scatter_wait(dd % NR, dd % NI)
        for dd in range(PEEL):
            idx_wait(NFULL + dd, (NFULL + dd) % NI)

        @pl.when(wid < NREST)
        def _():
            gl = NCHUNK - NREST + wid
            pltpu.sync_copy(gidx_hbm.at[gl], gi.at[0])
            pltpu.sync_copy(sidx_hbm.at[gl], si.at[0])
            pltpu.async_copy(src_hbm.at[gi.at[0]], rows[0], gsem[0]).wait()
            pltpu.sync_copy(rows[0], acc.at[si.at[0]], add=True)

        plsc.subcore_barrier()
        pltpu.sync_copy(acc.at[pl.ds(slab, rows_per_tile)],
                        out_hbm.at[c, pl.ds(slab, rows_per_tile)])

    return sc_pass


_sc_pass = _make_sc_pass(X_PAD)


def _mm_body(x_ref, t_ref, o_ref):
    o_ref[...] = jnp.dot(x_ref[...], t_ref[...],
                         preferred_element_type=jnp.float32)


def _matmul(Xpad, theta):
    blk = X_PAD // 10
    return pl.pallas_call(
        _mm_body,
        grid=(10,),
        in_specs=[pl.BlockSpec((blk, D), lambda i: (i, 0)),
                  pl.BlockSpec((D, D), lambda i: (0, 0))],
        out_specs=pl.BlockSpec((blk, D), lambda i: (i, 0)),
        out_shape=jax.ShapeDtypeStruct((X_PAD, D), jnp.float32),
    )(Xpad, theta)


def _sum2_body(p_ref, o_ref):
    o_ref[...] = p_ref[0] + p_ref[1]


def _sum_partials(P):
    n = P.shape[1]
    blk = n // 10
    return pl.pallas_call(
        _sum2_body,
        grid=(10,),
        in_specs=[pl.BlockSpec((NC, blk, D), lambda i: (0, i, 0))],
        out_specs=pl.BlockSpec((blk, D), lambda i: (i, 0)),
        out_shape=jax.ShapeDtypeStruct((n, D), jnp.float32),
    )(P)


def _sum2b_body(p_ref, b_ref, o_ref):
    o_ref[...] = p_ref[0] + p_ref[1] + b_ref[...]


def _sum_partials_bias(P, bias2d):
    n = P.shape[1]
    blk = n // 10
    return pl.pallas_call(
        _sum2b_body,
        grid=(10,),
        in_specs=[pl.BlockSpec((NC, blk, D), lambda i: (0, i, 0)),
                  pl.BlockSpec((1, D), lambda i: (0, 0))],
        out_specs=pl.BlockSpec((blk, D), lambda i: (i, 0)),
        out_shape=jax.ShapeDtypeStruct((n, D), jnp.float32),
    )(P, bias2d)


def kernel(X, node_idx, hyedge_idx, theta, bias):
    ni = node_idx.astype(jnp.int32).reshape(NCHUNK, CH)
    he = hyedge_idx.astype(jnp.int32).reshape(NCHUNK, CH)
    Xpad = jnp.pad(X, ((0, X_PAD - N_NODES), (0, 0)))
    Xp = _matmul(Xpad, theta)
    Yp = _sc_pass(Xp, ni, he)
    Y = _sum_partials(Yp)
    Xnp = _sc_pass(Y, he, ni)
    Xn = _sum_partials_bias(Xnp, bias.reshape(1, D))
    return Xn[:N_NODES]


# speedup vs baseline: 7.9875x; 1.0445x over previous
import functools
import math

import jax
import jax.numpy as jnp
from jax import lax
from jax.experimental import pallas as pl
from jax.experimental.pallas import tpu as pltpu
from jax.experimental.pallas import tpu_sc as plsc

N_NODES = 10000
N_HYEDGES = 5000
N_INC = 320000
D = 128

NC = 2
NS = 16
NW = NC * NS
CH = 64
NCHUNK = N_INC // CH
NFULL = NCHUNK // NW
NREST = NCHUNK - NFULL * NW

Y_PAD = 5120
X_PAD = 10240


def _make_sc_pass(n_dst_pad, nr, ni, ga, pf=4):
    assert ga < nr and pf <= ni - nr
    rows_per_tile = n_dst_pad // NS
    unroll = math.lcm(ni, nr)
    steady = NFULL - nr
    n_loop = steady // unroll
    trail = steady - n_loop * unroll
    mesh = plsc.VectorSubcoreMesh(core_axis_name="c", subcore_axis_name="s")

    @functools.partial(
        pl.kernel,
        out_type=jax.ShapeDtypeStruct((NC, n_dst_pad, D), jnp.float32),
        mesh=mesh,
        scratch_types=(
            [pltpu.VMEM((16, D), jnp.float32),
             pltpu.VMEM((ni, CH), jnp.int32),
             pltpu.VMEM((ni, CH), jnp.int32)]
            + [pltpu.VMEM((CH, D), jnp.float32) for _ in range(nr)]
            + [pltpu.VMEM_SHARED((n_dst_pad, D), jnp.float32)]
            + [pltpu.SemaphoreType.DMA for _ in range(ni + 2 * nr)]
        ),
    )
    def sc_pass(src_hbm, gidx_hbm, sidx_hbm, out_hbm, zb, gi, si, *rest):
        rows = rest[:nr]
        acc = rest[nr]
        isem = rest[nr + 1:nr + 1 + ni]
        gsem = rest[nr + 1 + ni:nr + 1 + ni + nr]
        ssem = rest[nr + 1 + ni + nr:nr + 1 + ni + 2 * nr]
        c = lax.axis_index("c")
        s = lax.axis_index("s")
        wid = s * NC + c

        def zrow(i, _):
            def zcol(j, _):
                zb[i, pl.ds(j * 16, 16)] = jnp.zeros((16,), jnp.float32)
                return 0
            return lax.fori_loop(0, D // 16, zcol, 0)
        lax.fori_loop(0, 16, zrow, 0)

        slab = s * rows_per_tile

        def zslab(k, _):
            pltpu.sync_copy(zb, acc.at[pl.ds(slab + k * 16, 16)])
            return 0
        lax.fori_loop(0, rows_per_tile // 16, zslab, 0)
        plsc.subcore_barrier()

        def chunk_of(j):
            return jnp.minimum(wid + NW * j, NCHUNK - 1)

        def idx_issue(j, sl):
            g = chunk_of(j)
            pltpu.async_copy(gidx_hbm.at[g], gi.at[sl], isem[sl])
            pltpu.async_copy(sidx_hbm.at[g], si.at[sl], isem[sl])

        def idx_wait(j, sl):
            g = chunk_of(j)
            pltpu.make_async_copy(gidx_hbm.at[g], gi.at[sl], isem[sl]).wait()
            pltpu.make_async_copy(sidx_hbm.at[g], si.at[sl], isem[sl]).wait()

        def gather_issue(rs, isl):
            pltpu.async_copy(src_hbm.at[gi.at[isl]], rows[rs], gsem[rs])

        def gather_wait(rs, isl):
            pltpu.make_async_copy(src_hbm.at[gi.at[isl]], rows[rs],
                                  gsem[rs]).wait()

        def scatter_issue(rs, isl):
            pltpu.async_copy(rows[rs], acc.at[si.at[isl]], ssem[rs], add=True)

        def scatter_wait(rs, isl):
            pltpu.make_async_copy(rows[rs], acc.at[si.at[isl]],
                                  ssem[rs]).wait()

        def steps(j, u):
            if u >= nr:
                scatter_wait(u % nr, (u + ni - nr) % ni)
            idx_issue(j + pf, (u + pf) % ni)
            idx_wait(j, u % ni)
            gather_issue(u % nr, u % ni)
            if u >= ga:
                gather_wait((u - ga) % nr, (u - ga) % ni)
                scatter_issue((u - ga) % nr, (u - ga) % ni)

        for jj in range(pf):
            idx_issue(jj, jj)
        for jj in range(nr):
            steps(jj, jj)

        def body(k, _):
            j0 = nr + k * unroll
            for v in range(unroll):
                steps(j0 + v, nr + v)
            return 0

        lax.fori_loop(0, n_loop, body, 0)

        for v in range(trail):
            steps(nr + n_loop * unroll + v, nr + v)

        for d in range(NFULL - ga, NFULL):
            gather_wait(d % nr, d % ni)
            scatter_issue(d % nr, d % ni)
        for d in range(NFULL - nr, NFULL):
            scatter_wait(d % nr, d % ni)
        for d in range(pf):
            idx_wait(NFULL + d, (NFULL + d) % ni)

        @pl.when(wid < NREST)
        def _():
            gl = NCHUNK - NREST + wid
            pltpu.sync_copy(gidx_hbm.at[gl], gi.at[0])
            pltpu.sync_copy(sidx_hbm.at[gl], si.at[0])
            pltpu.async_copy(src_hbm.at[gi.at[0]], rows[0], gsem[0]).wait()
            pltpu.sync_copy(rows[0], acc.at[si.at[0]], add=True)

        plsc.subcore_barrier()
        pltpu.sync_copy(acc.at[pl.ds(slab, rows_per_tile)],
                        out_hbm.at[c, pl.ds(slab, rows_per_tile)])

    return sc_pass


_sc_pass_A = _make_sc_pass(Y_PAD, nr=6, ni=12, ga=3)
_sc_pass_B = _make_sc_pass(X_PAD, nr=4, ni=8, ga=2)


def _mm_body(x_ref, t_ref, o_ref):
    o_ref[...] = jnp.dot(x_ref[...], t_ref[...],
                         preferred_element_type=jnp.float32)


def _matmul(X, theta):
    return pl.pallas_call(
        _mm_body,
        grid=(10,),
        in_specs=[pl.BlockSpec((1000, D), lambda i: (i, 0)),
                  pl.BlockSpec((D, D), lambda i: (0, 0))],
        out_specs=pl.BlockSpec((1000, D), lambda i: (i, 0)),
        out_shape=jax.ShapeDtypeStruct((N_NODES, D), jnp.float32),
    )(X, theta)


def _sum2_body(p_ref, o_ref):
    o_ref[...] = p_ref[0] + p_ref[1]


def _sum_partials(P):
    n = P.shape[1]
    blk = n // 10
    return pl.pallas_call(
        _sum2_body,
        grid=(10,),
        in_specs=[pl.BlockSpec((NC, blk, D), lambda i: (0, i, 0))],
        out_specs=pl.BlockSpec((blk, D), lambda i: (i, 0)),
        out_shape=jax.ShapeDtypeStruct((n, D), jnp.float32),
    )(P)


def _sum2b_body(p_ref, b_ref, o_ref):
    o_ref[...] = p_ref[0] + p_ref[1] + b_ref[...]


def _sum_partials_bias(P, bias2d):
    n = P.shape[1]
    blk = n // 10
    return pl.pallas_call(
        _sum2b_body,
        grid=(10,),
        in_specs=[pl.BlockSpec((NC, blk, D), lambda i: (0, i, 0)),
                  pl.BlockSpec((1, D), lambda i: (0, 0))],
        out_specs=pl.BlockSpec((blk, D), lambda i: (i, 0)),
        out_shape=jax.ShapeDtypeStruct((n, D), jnp.float32),
    )(P, bias2d)


def kernel(X, node_idx, hyedge_idx, theta, bias):
    ni_ = node_idx.astype(jnp.int32).reshape(NCHUNK, CH)
    he = hyedge_idx.astype(jnp.int32).reshape(NCHUNK, CH)
    Xp = _matmul(X, theta)
    Yp = _sc_pass_A(Xp, ni_, he)
    Y = _sum_partials(Yp)
    Xnp = _sc_pass_B(Y, he, ni_)
    Xn = _sum_partials_bias(Xnp, bias.reshape(1, D))
    return Xn[:N_NODES]
